# baseline (speedup 1.0000x reference)
import functools

import jax
import jax.numpy as jnp
from jax import lax
from jax.experimental import pallas as pl
from jax.experimental.pallas import tpu as pltpu
from jax.experimental.pallas import tpu_sc as plsc

_NC = 2
_NS = 16
_CH = 128


def _make_sc_gather(B, D_G, D_M):
    NW = _NC * _NS
    bpw = B // NW
    nch = bpw // _CH

    mesh = plsc.VectorSubcoreMesh(core_axis_name="c", subcore_axis_name="s")

    @functools.partial(
        pl.kernel,
        out_type=[
            jax.ShapeDtypeStruct((B, D_G), jnp.float32),
            jax.ShapeDtypeStruct((B, D_G), jnp.float32),
            jax.ShapeDtypeStruct((B, D_M), jnp.float32),
            jax.ShapeDtypeStruct((B, D_M), jnp.float32),
        ],
        mesh=mesh,
        compiler_params=pltpu.CompilerParams(use_tc_tiling_on_sc=False),
        scratch_types=[
            pltpu.VMEM((nch, _CH), jnp.int32),
            pltpu.VMEM((nch, _CH), jnp.int32),
            pltpu.VMEM((nch, _CH, D_G), jnp.float32),
            pltpu.VMEM((nch, _CH, D_G), jnp.float32),
            pltpu.VMEM((nch, _CH, D_M), jnp.float32),
            pltpu.VMEM((nch, _CH, D_M), jnp.float32),
            pltpu.SemaphoreType.DMA,
            pltpu.SemaphoreType.DMA,
        ],
    )
    def gather_k(uidx_h, iidx_h, ug_h, ig_h, um_h, im_h,
                 ug_o, ig_o, um_o, im_o,
                 uidx_v, iidx_v, ug_v, ig_v, um_v, im_v, gsem, wsem):
        wid = lax.axis_index("s") * _NC + lax.axis_index("c")
        base = wid * bpw
        for j in range(nch):
            pltpu.sync_copy(uidx_h.at[pl.ds(base + j * _CH, _CH)], uidx_v.at[j])
            pltpu.sync_copy(iidx_h.at[pl.ds(base + j * _CH, _CH)], iidx_v.at[j])
        gathers = []
        for j in range(nch):
            gathers.append(pltpu.async_copy(ug_h.at[uidx_v.at[j]], ug_v.at[j], gsem))
            gathers.append(pltpu.async_copy(ig_h.at[iidx_v.at[j]], ig_v.at[j], gsem))
            gathers.append(pltpu.async_copy(um_h.at[uidx_v.at[j]], um_v.at[j], gsem))
            gathers.append(pltpu.async_copy(im_h.at[iidx_v.at[j]], im_v.at[j], gsem))
        for g in gathers:
            g.wait()
        writes = []
        for j in range(nch):
            sl = pl.ds(base + j * _CH, _CH)
            writes.append(pltpu.async_copy(ug_v.at[j], ug_o.at[sl], wsem))
            writes.append(pltpu.async_copy(ig_v.at[j], ig_o.at[sl], wsem))
            writes.append(pltpu.async_copy(um_v.at[j], um_o.at[sl], wsem))
            writes.append(pltpu.async_copy(im_v.at[j], im_o.at[sl], wsem))
        for w in writes:
            w.wait()

    return gather_k


def _mlp_body(ug_r, ig_r, um_r, im_r, w1u_r, w1i_r, b1_r, w2_r, b2_r,
              w3_r, b3_r, wpg_r, wpm_r, bp_r, o_r):
    dn = (((1,), (1,)), ((), ()))
    h = jnp.maximum(
        lax.dot_general(um_r[...], w1u_r[...], dn, preferred_element_type=jnp.float32)
        + lax.dot_general(im_r[...], w1i_r[...], dn, preferred_element_type=jnp.float32)
        + b1_r[...], 0.0)
    h = jnp.maximum(
        lax.dot_general(h, w2_r[...], dn, preferred_element_type=jnp.float32)
        + b2_r[...], 0.0)
    h = jnp.maximum(
        lax.dot_general(h, w3_r[...], dn, preferred_element_type=jnp.float32)
        + b3_r[...], 0.0)
    gmf = ug_r[...] * ig_r[...]
    logit = (jnp.sum(gmf * wpg_r[...], axis=1)
             + jnp.sum(h * wpm_r[...], axis=1) + bp_r[0, 0])
    o_r[...] = jax.nn.sigmoid(logit)


def _make_tc_mlp(B, D_G, D_M, H1, H2, H3, BLK=512):
    nblk = B // BLK
    full = lambda r, c: pl.BlockSpec((r, c), lambda i: (0, 0))
    return pl.pallas_call(
        _mlp_body,
        grid=(nblk,),
        in_specs=[
            pl.BlockSpec((BLK, D_G), lambda i: (i, 0)),
            pl.BlockSpec((BLK, D_G), lambda i: (i, 0)),
            pl.BlockSpec((BLK, D_M), lambda i: (i, 0)),
            pl.BlockSpec((BLK, D_M), lambda i: (i, 0)),
            full(H1, D_M),
            full(H1, D_M),
            full(1, H1),
            full(H2, H1),
            full(1, H2),
            full(H3, H2),
            full(1, H3),
            full(1, D_G),
            full(1, H3),
            pl.BlockSpec(memory_space=pltpu.SMEM),
        ],
        out_specs=pl.BlockSpec((BLK,), lambda i: (i,)),
        out_shape=jax.ShapeDtypeStruct((B,), jnp.float32),
    )


def kernel(user_indices, item_indices, user_gmf, item_gmf, user_mlp, item_mlp,
           W1, b1, W2, b2, W3, b3, Wp, bp):
    B = user_indices.shape[0]
    D_G = user_gmf.shape[1]
    D_M = user_mlp.shape[1]
    H1, H2, H3 = W1.shape[0], W2.shape[0], W3.shape[0]

    ui = user_indices.astype(jnp.int32)
    ii = item_indices.astype(jnp.int32)
    ug, ig, um, im = _make_sc_gather(B, D_G, D_M)(
        ui, ii, user_gmf, item_gmf, user_mlp, item_mlp)

    mlp = _make_tc_mlp(B, D_G, D_M, H1, H2, H3)
    return mlp(ug, ig, um, im,
               W1[:, :D_M], W1[:, D_M:], b1.reshape(1, H1),
               W2, b2.reshape(1, H2), W3, b3.reshape(1, H3),
               Wp[:, :D_G], Wp[:, D_G:], bp.reshape(1, 1))


# speedup vs baseline: 1.1067x; 1.1067x over previous
import functools

import jax
import jax.numpy as jnp
from jax import lax
from jax.experimental import pallas as pl
from jax.experimental.pallas import tpu as pltpu
from jax.experimental.pallas import tpu_sc as plsc

_NC = 2
_NS = 16
_CH = 128


def _make_sc_gather(B, D_G, D_M):
    NW = _NC * _NS
    bpw = B // NW
    nch = bpw // _CH

    mesh = plsc.VectorSubcoreMesh(core_axis_name="c", subcore_axis_name="s")

    @functools.partial(
        pl.kernel,
        out_type=[
            jax.ShapeDtypeStruct((B, 128), jnp.float32),
            jax.ShapeDtypeStruct((B, 128), jnp.float32),
        ],
        mesh=mesh,
        compiler_params=pltpu.CompilerParams(use_tc_tiling_on_sc=False),
        scratch_types=[
            pltpu.VMEM((nch, _CH), jnp.int32),
            pltpu.VMEM((nch, _CH), jnp.int32),
            pltpu.VMEM((nch, _CH, D_G), jnp.float32),
            pltpu.VMEM((nch, _CH, D_G), jnp.float32),
            pltpu.VMEM((nch, _CH, D_M), jnp.float32),
            pltpu.VMEM((nch, _CH, D_M), jnp.float32),
            pltpu.SemaphoreType.DMA,
            pltpu.SemaphoreType.DMA,
        ],
    )
    def gather_k(uidx_h, iidx_h, ug_h, ig_h, um_h, im_h,
                 x_o, g_o,
                 uidx_v, iidx_v, ug_v, ig_v, um_v, im_v, gsem, wsem):
        wid = lax.axis_index("s") * _NC + lax.axis_index("c")
        base = wid * bpw
        for j in range(nch):
            pltpu.sync_copy(uidx_h.at[pl.ds(base + j * _CH, _CH)], uidx_v.at[j])
            pltpu.sync_copy(iidx_h.at[pl.ds(base + j * _CH, _CH)], iidx_v.at[j])
        gathers = []
        for j in range(nch):
            gathers.append(pltpu.async_copy(ug_h.at[uidx_v.at[j]], ug_v.at[j], gsem))
            gathers.append(pltpu.async_copy(ig_h.at[iidx_v.at[j]], ig_v.at[j], gsem))
            gathers.append(pltpu.async_copy(um_h.at[uidx_v.at[j]], um_v.at[j], gsem))
            gathers.append(pltpu.async_copy(im_h.at[iidx_v.at[j]], im_v.at[j], gsem))
        for g in gathers:
            g.wait()
        writes = []
        for j in range(nch):
            sl = pl.ds(base + j * _CH, _CH)
            writes.append(pltpu.async_copy(um_v.at[j], x_o.at[sl, pl.ds(0, D_M)], wsem))
            writes.append(pltpu.async_copy(im_v.at[j], x_o.at[sl, pl.ds(D_M, D_M)], wsem))
            writes.append(pltpu.async_copy(ug_v.at[j], g_o.at[sl, pl.ds(0, D_G)], wsem))
            writes.append(pltpu.async_copy(ig_v.at[j], g_o.at[sl, pl.ds(D_G, D_G)], wsem))
        for w in writes:
            w.wait()

    return gather_k


def _mlp_body(D_G, x_r, g_r, w1_r, b1_r, w2_r, b2_r,
              w3_r, b3_r, wpg_r, wpm_r, bp_r, o_r):
    dn = (((1,), (1,)), ((), ()))
    h = jnp.maximum(
        lax.dot_general(x_r[...], w1_r[...], dn, preferred_element_type=jnp.float32)
        + b1_r[...], 0.0)
    h = jnp.maximum(
        lax.dot_general(h, w2_r[...], dn, preferred_element_type=jnp.float32)
        + b2_r[...], 0.0)
    h = jnp.maximum(
        lax.dot_general(h, w3_r[...], dn, preferred_element_type=jnp.float32)
        + b3_r[...], 0.0)
    g = g_r[...]
    gmf = g[:, :D_G] * g[:, D_G:2 * D_G]
    logit = (jnp.sum(gmf * wpg_r[...], axis=1)
             + jnp.sum(h * wpm_r[...], axis=1) + bp_r[0, 0])
    o_r[...] = jax.nn.sigmoid(logit)


def _make_tc_mlp(B, D_G, H1, H2, H3, BLK=512):
    nblk = B // BLK
    full = lambda r, c: pl.BlockSpec((r, c), lambda i: (0, 0))
    return pl.pallas_call(
        functools.partial(_mlp_body, D_G),
        grid=(nblk,),
        in_specs=[
            pl.BlockSpec((BLK, 128), lambda i: (i, 0)),
            pl.BlockSpec((BLK, 128), lambda i: (i, 0)),
            full(H1, 128),
            full(1, H1),
            full(H2, H1),
            full(1, H2),
            full(H3, H2),
            full(1, H3),
            full(1, D_G),
            full(1, H3),
            pl.BlockSpec(memory_space=pltpu.SMEM),
        ],
        out_specs=pl.BlockSpec((BLK,), lambda i: (i,)),
        out_shape=jax.ShapeDtypeStruct((B,), jnp.float32),
    )


def kernel(user_indices, item_indices, user_gmf, item_gmf, user_mlp, item_mlp,
           W1, b1, W2, b2, W3, b3, Wp, bp):
    B = user_indices.shape[0]
    D_G = user_gmf.shape[1]
    D_M = user_mlp.shape[1]
    H1, H2, H3 = W1.shape[0], W2.shape[0], W3.shape[0]

    ui = user_indices.astype(jnp.int32)
    ii = item_indices.astype(jnp.int32)
    xcat, gcat = _make_sc_gather(B, D_G, D_M)(
        ui, ii, user_gmf, item_gmf, user_mlp, item_mlp)

    mlp = _make_tc_mlp(B, D_G, H1, H2, H3)
    return mlp(xcat, gcat,
               W1, b1.reshape(1, H1),
               W2, b2.reshape(1, H2), W3, b3.reshape(1, H3),
               Wp[:, :D_G], Wp[:, D_G:], bp.reshape(1, 1))


# speedup vs baseline: 1.3327x; 1.2042x over previous
import functools

import jax
import jax.numpy as jnp
from jax import lax
from jax.experimental import pallas as pl
from jax.experimental.pallas import tpu as pltpu
from jax.experimental.pallas import tpu_sc as plsc

_NC = 2
_NS = 16
_CH = 128


def _make_sc_gather(B, D_G, D_M):
    NW = _NC * _NS
    bpw = B // NW
    nch = bpw // _CH

    mesh = plsc.VectorSubcoreMesh(core_axis_name="c", subcore_axis_name="s")

    @functools.partial(
        pl.kernel,
        out_type=[
            jax.ShapeDtypeStruct((B, D_G), jnp.float32),
            jax.ShapeDtypeStruct((B, D_G), jnp.float32),
            jax.ShapeDtypeStruct((B, D_M), jnp.float32),
            jax.ShapeDtypeStruct((B, D_M), jnp.float32),
        ],
        mesh=mesh,
        scratch_types=[
            pltpu.SMEM((bpw,), jnp.int32),
            pltpu.SMEM((bpw,), jnp.int32),
            pltpu.VMEM((bpw,), jnp.int32),
            pltpu.VMEM((bpw,), jnp.int32),
            pltpu.VMEM((_CH, D_G), jnp.float32),
            pltpu.VMEM((_CH, D_G), jnp.float32),
            pltpu.VMEM((_CH, D_M), jnp.float32),
            pltpu.VMEM((_CH, D_M), jnp.float32),
            pltpu.SemaphoreType.DMA,
            pltpu.SemaphoreType.DMA,
        ],
    )
    def gather_k(uidx_h, iidx_h, ug_h, ig_h, um_h, im_h,
                 ug_o, ig_o, um_o, im_o,
                 uidx_s, iidx_s, uidx_v, iidx_v, ug_v, ig_v, um_v, im_v,
                 gsem, wsem):
        wid = lax.axis_index("s") * _NC + lax.axis_index("c")
        base = wid * bpw
        pltpu.sync_copy(uidx_h.at[pl.ds(base, bpw)], uidx_v)
        pltpu.sync_copy(iidx_h.at[pl.ds(base, bpw)], iidx_v)
        for c in range(nch):
            row0 = base + c * _CH

            def fetch(g, _, c=c):
                xu = uidx_v[pl.ds(c * _CH + g * 16, 16)]
                xi = iidx_v[pl.ds(c * _CH + g * 16, 16)]
                for k in range(16):
                    i = g * 16 + k
                    ru = xu[k]
                    ri = xi[k]
                    pltpu.async_copy(ug_h.at[ru], ug_v.at[i], gsem)
                    pltpu.async_copy(ig_h.at[ri], ig_v.at[i], gsem)
                    pltpu.async_copy(um_h.at[ru], um_v.at[i], gsem)
                    pltpu.async_copy(im_h.at[ri], im_v.at[i], gsem)
                return 0

            lax.fori_loop(0, _CH // 16, fetch, 0)
            pltpu.make_async_copy(ug_h.at[pl.ds(0, _CH)], ug_v, gsem).wait()
            pltpu.make_async_copy(ig_h.at[pl.ds(0, _CH)], ig_v, gsem).wait()
            pltpu.make_async_copy(um_h.at[pl.ds(0, _CH)], um_v, gsem).wait()
            pltpu.make_async_copy(im_h.at[pl.ds(0, _CH)], im_v, gsem).wait()
            sl = pl.ds(row0, _CH)
            writes = [
                pltpu.async_copy(um_v, um_o.at[sl], wsem),
                pltpu.async_copy(im_v, im_o.at[sl], wsem),
                pltpu.async_copy(ug_v, ug_o.at[sl], wsem),
                pltpu.async_copy(ig_v, ig_o.at[sl], wsem),
            ]
            for w in writes:
                w.wait()

    return gather_k


def _mlp_body(D_G, ug_r, ig_r, um_r, im_r, w1u_r, w1i_r, b1_r, w2_r, b2_r,
              w3_r, b3_r, wpg_r, wpm_r, bp_r, o_r):
    dn = (((1,), (1,)), ((), ()))
    h = jnp.maximum(
        lax.dot_general(um_r[...], w1u_r[...], dn, preferred_element_type=jnp.float32)
        + lax.dot_general(im_r[...], w1i_r[...], dn, preferred_element_type=jnp.float32)
        + b1_r[...], 0.0)
    h = jnp.maximum(
        lax.dot_general(h, w2_r[...], dn, preferred_element_type=jnp.float32)
        + b2_r[...], 0.0)
    h = jnp.maximum(
        lax.dot_general(h, w3_r[...], dn, preferred_element_type=jnp.float32)
        + b3_r[...], 0.0)
    gmf = ug_r[...] * ig_r[...]
    logit = (jnp.sum(gmf * wpg_r[...], axis=1)
             + jnp.sum(h * wpm_r[...], axis=1) + bp_r[0, 0])
    o_r[...] = jax.nn.sigmoid(logit)


def _make_tc_mlp(B, D_G, D_M, H1, H2, H3, BLK=512):
    nblk = B // BLK
    full = lambda r, c: pl.BlockSpec((r, c), lambda i: (0, 0))
    return pl.pallas_call(
        functools.partial(_mlp_body, D_G),
        grid=(nblk,),
        in_specs=[
            pl.BlockSpec((BLK, D_G), lambda i: (i, 0)),
            pl.BlockSpec((BLK, D_G), lambda i: (i, 0)),
            pl.BlockSpec((BLK, D_M), lambda i: (i, 0)),
            pl.BlockSpec((BLK, D_M), lambda i: (i, 0)),
            full(H1, D_M),
            full(H1, D_M),
            full(1, H1),
            full(H2, H1),
            full(1, H2),
            full(H3, H2),
            full(1, H3),
            full(1, D_G),
            full(1, H3),
            pl.BlockSpec(memory_space=pltpu.SMEM),
        ],
        out_specs=pl.BlockSpec((BLK,), lambda i: (i,)),
        out_shape=jax.ShapeDtypeStruct((B,), jnp.float32),
    )


def kernel(user_indices, item_indices, user_gmf, item_gmf, user_mlp, item_mlp,
           W1, b1, W2, b2, W3, b3, Wp, bp):
    B = user_indices.shape[0]
    D_G = user_gmf.shape[1]
    D_M = user_mlp.shape[1]
    H1, H2, H3 = W1.shape[0], W2.shape[0], W3.shape[0]

    ui = user_indices.astype(jnp.int32)
    ii = item_indices.astype(jnp.int32)
    ug, ig, um, im = _make_sc_gather(B, D_G, D_M)(
        ui, ii, user_gmf, item_gmf, user_mlp, item_mlp)

    mlp = _make_tc_mlp(B, D_G, D_M, H1, H2, H3)
    return mlp(ug, ig, um, im,
               W1[:, :D_M], W1[:, D_M:], b1.reshape(1, H1),
               W2, b2.reshape(1, H2), W3, b3.reshape(1, H3),
               Wp[:, :D_G], Wp[:, D_G:], bp.reshape(1, 1))
